# Initial kernel scaffold; baseline (speedup 1.0000x reference)
import functools

import jax
import jax.numpy as jnp
from jax.experimental import pallas as pl


def _stage_kernel(nlayers, c1, acts, *refs):
    q_ref, kt_ref, p1_ref, p2_ref = refs[:4]
    wrefs = refs[4:4 + 2 * nlayers]
    out_ref = refs[-1]

    t = q_ref.shape[1]
    n2 = kt_ref.shape[2]

    d = None
    for c in range(3):
        diff = q_ref[0, :, c:c + 1] - kt_ref[0, c:c + 1, :]
        d = diff * diff if d is None else d + diff * diff

    iota = jax.lax.broadcasted_iota(jnp.int32, (t, n2), 1)
    big = jnp.float32(3.0e38)
    dw = d
    picks = []
    for k in range(3):
        mn = jnp.min(dw, axis=1, keepdims=True)
        am = jnp.min(jnp.where(dw == mn, iota, n2), axis=1, keepdims=True)
        oh = iota == am
        picks.append((mn, oh))
        if k < 2:
            dw = jnp.where(oh, big, dw)

    ws = [1.0 / jnp.maximum(mn, 1e-10) for mn, _ in picks]
    wtot = ws[0] + ws[1] + ws[2]
    wmat = None
    for wk, (_, oh) in zip(ws, picks):
        term = jnp.where(oh, wk / wtot, 0.0)
        wmat = term if wmat is None else wmat + term

    interp = jnp.dot(wmat, p2_ref[0, :, :], preferred_element_type=jnp.float32)

    w0 = wrefs[0]
    b0 = wrefs[1]
    x = (jnp.dot(p1_ref[0, :, :], w0[:c1, :], preferred_element_type=jnp.float32)
         + jnp.dot(interp, w0[c1:, :], preferred_element_type=jnp.float32)
         + b0[0:1, :])
    x = jnp.maximum(x, 0.0) if acts[0] == 'r' else jnp.tanh(x)
    for i in range(1, nlayers):
        w = wrefs[2 * i]
        b = wrefs[2 * i + 1]
        x = jnp.dot(x, w, preferred_element_type=jnp.float32) + b[0:1, :]
        x = jnp.maximum(x, 0.0) if acts[i] == 'r' else jnp.tanh(x)
    out_ref[0, :, :] = x


def _fp_stage(xyz1, xyz2, points1, points2, layers, acts, tile):
    bb, n1, _ = xyz1.shape
    n2 = xyz2.shape[1]
    c1 = points1.shape[2]
    c2 = points2.shape[2]
    out_dim = layers[-1][0].shape[1]
    kt = jnp.swapaxes(xyz2, 1, 2)

    args = [xyz1, kt, points1, points2]
    in_specs = [
        pl.BlockSpec((1, tile, 3), lambda b, s: (b, s, 0)),
        pl.BlockSpec((1, 3, n2), lambda b, s: (b, 0, 0)),
        pl.BlockSpec((1, tile, c1), lambda b, s: (b, s, 0)),
        pl.BlockSpec((1, n2, c2), lambda b, s: (b, 0, 0)),
    ]
    for w, b in layers:
        args += [w, b.reshape(1, -1)]
        in_specs.append(pl.BlockSpec(w.shape, lambda b, s: (0, 0)))
        in_specs.append(pl.BlockSpec((1, w.shape[1]), lambda b, s: (0, 0)))

    return pl.pallas_call(
        functools.partial(_stage_kernel, len(layers), c1, acts),
        grid=(bb, n1 // tile),
        in_specs=in_specs,
        out_specs=pl.BlockSpec((1, tile, out_dim), lambda b, s: (b, s, 0)),
        out_shape=jax.ShapeDtypeStruct((bb, n1, out_dim), jnp.float32),
    )(*args)


def kernel(xyz0, xyz1, xyz2, xyz3, xyz4, points0, points1, points2, points3,
           points4, fa1_W0, fa1_b0, fa1_W1, fa1_b1, fa2_W0, fa2_b0, fa2_W1,
           fa2_b1, fa3_W0, fa3_b0, fa3_W1, fa3_b1, fa4_W0, fa4_b0, fa4_W1,
           fa4_b1, fa4_W2, fa4_b2, head_W0, head_b0, head_W1, head_b1,
           head_W2, head_b2):
    p3 = _fp_stage(xyz3, xyz4, points3, points4,
                   [(fa1_W0, fa1_b0), (fa1_W1, fa1_b1)], 'rr', 64)
    p2 = _fp_stage(xyz2, xyz3, points2, p3,
                   [(fa2_W0, fa2_b0), (fa2_W1, fa2_b1)], 'rr', 256)
    p1 = _fp_stage(xyz1, xyz2, points1, p2,
                   [(fa3_W0, fa3_b0), (fa3_W1, fa3_b1)], 'rr', 1024)
    head_W2p = jnp.pad(head_W2, ((0, 0), (0, 5)))
    head_b2p = jnp.pad(head_b2, (0, 5))
    out8 = _fp_stage(xyz0, xyz1, points0, p1,
                     [(fa4_W0, fa4_b0), (fa4_W1, fa4_b1), (fa4_W2, fa4_b2),
                      (head_W0, head_b0), (head_W1, head_b1),
                      (head_W2p, head_b2p)], 'rrrrrt', 512)
    return out8[..., :3]


# speedup vs baseline: 17.1051x; 17.1051x over previous
import functools

import jax
import jax.numpy as jnp
from jax.experimental import pallas as pl


def _mm(x, w):
    return jnp.dot(x.astype(jnp.bfloat16), w.astype(jnp.bfloat16),
                   preferred_element_type=jnp.float32)


def _stage_kernel(nlayers, c1, acts, *refs):
    q_ref, kt_ref, p1_ref, p2_ref = refs[:4]
    wrefs = refs[4:4 + 2 * nlayers]
    out_ref = refs[-1]

    t = q_ref.shape[1]
    n2 = kt_ref.shape[2]

    q = q_ref[0, :, :]
    kt = kt_ref[0, :, :]
    aa = jnp.sum(q * q, axis=1, keepdims=True)
    bb = jnp.sum(kt * kt, axis=0, keepdims=True)
    ab = _mm(q, kt)
    d = jnp.maximum(aa + bb - 2.0 * ab, 0.0)

    iota = jax.lax.broadcasted_iota(jnp.int32, (t, n2), 1)
    big = jnp.float32(3.0e38)
    dw = d
    picks = []
    for k in range(3):
        mn = jnp.min(dw, axis=1, keepdims=True)
        am = jnp.min(jnp.where(dw == mn, iota, n2), axis=1, keepdims=True)
        oh = iota == am
        picks.append((mn, oh))
        if k < 2:
            dw = jnp.where(oh, big, dw)

    ws = [1.0 / jnp.maximum(mn, 1e-10) for mn, _ in picks]
    wtot = ws[0] + ws[1] + ws[2]
    wmat = None
    for wk, (_, oh) in zip(ws, picks):
        term = jnp.where(oh, wk / wtot, 0.0)
        wmat = term if wmat is None else wmat + term

    interp = jnp.dot(wmat, p2_ref[0, :, :], preferred_element_type=jnp.float32, precision=jax.lax.Precision.HIGHEST)

    w0 = wrefs[0]
    b0 = wrefs[1]
    x = (_mm(p1_ref[0, :, :], w0[:c1, :]) + _mm(interp, w0[c1:, :])
         + b0[0:1, :])
    x = jnp.maximum(x, 0.0) if acts[0] == 'r' else jnp.tanh(x)
    for i in range(1, nlayers):
        w = wrefs[2 * i]
        b = wrefs[2 * i + 1]
        x = _mm(x, w[:, :]) + b[0:1, :]
        x = jnp.maximum(x, 0.0) if acts[i] == 'r' else jnp.tanh(x)
    out_ref[0, :, :] = x


def _fp_stage(xyz1, xyz2, points1, points2, layers, acts, tile):
    bb, n1, _ = xyz1.shape
    n2 = xyz2.shape[1]
    c1 = points1.shape[2]
    c2 = points2.shape[2]
    out_dim = layers[-1][0].shape[1]
    kt = jnp.swapaxes(xyz2, 1, 2)

    args = [xyz1, kt, points1, points2]
    in_specs = [
        pl.BlockSpec((1, tile, 3), lambda b, s: (b, s, 0)),
        pl.BlockSpec((1, 3, n2), lambda b, s: (b, 0, 0)),
        pl.BlockSpec((1, tile, c1), lambda b, s: (b, s, 0)),
        pl.BlockSpec((1, n2, c2), lambda b, s: (b, 0, 0)),
    ]
    for w, b in layers:
        args += [w, b.reshape(1, -1)]
        in_specs.append(pl.BlockSpec(w.shape, lambda b, s: (0, 0)))
        in_specs.append(pl.BlockSpec((1, w.shape[1]), lambda b, s: (0, 0)))

    return pl.pallas_call(
        functools.partial(_stage_kernel, len(layers), c1, acts),
        grid=(bb, n1 // tile),
        in_specs=in_specs,
        out_specs=pl.BlockSpec((1, tile, out_dim), lambda b, s: (b, s, 0)),
        out_shape=jax.ShapeDtypeStruct((bb, n1, out_dim), jnp.float32),
    )(*args)


def kernel(xyz0, xyz1, xyz2, xyz3, xyz4, points0, points1, points2, points3,
           points4, fa1_W0, fa1_b0, fa1_W1, fa1_b1, fa2_W0, fa2_b0, fa2_W1,
           fa2_b1, fa3_W0, fa3_b0, fa3_W1, fa3_b1, fa4_W0, fa4_b0, fa4_W1,
           fa4_b1, fa4_W2, fa4_b2, head_W0, head_b0, head_W1, head_b1,
           head_W2, head_b2):
    p3 = _fp_stage(xyz3, xyz4, points3, points4,
                   [(fa1_W0, fa1_b0), (fa1_W1, fa1_b1)], 'rr', 64)
    p2 = _fp_stage(xyz2, xyz3, points2, p3,
                   [(fa2_W0, fa2_b0), (fa2_W1, fa2_b1)], 'rr', 256)
    p1 = _fp_stage(xyz1, xyz2, points1, p2,
                   [(fa3_W0, fa3_b0), (fa3_W1, fa3_b1)], 'rr', 1024)
    head_W2p = jnp.pad(head_W2, ((0, 0), (0, 5)))
    head_b2p = jnp.pad(head_b2, (0, 5))
    out8 = _fp_stage(xyz0, xyz1, points0, p1,
                     [(fa4_W0, fa4_b0), (fa4_W1, fa4_b1), (fa4_W2, fa4_b2),
                      (head_W0, head_b0), (head_W1, head_b1),
                      (head_W2p, head_b2p)], 'rrrrrt', 512)
    return out8[..., :3]


# speedup vs baseline: 22.0970x; 1.2918x over previous
import functools

import jax
import jax.numpy as jnp
from jax.experimental import pallas as pl
from jax.experimental.pallas import tpu as pltpu


def _mm(x, w):
    return jnp.dot(x.astype(jnp.bfloat16), w.astype(jnp.bfloat16),
                   preferred_element_type=jnp.float32)


def _mm3(a, b):
    ah = a.astype(jnp.bfloat16)
    al = (a - ah.astype(jnp.float32)).astype(jnp.bfloat16)
    bh = b.astype(jnp.bfloat16)
    bl = (b - bh.astype(jnp.float32)).astype(jnp.bfloat16)
    f32 = jnp.float32
    return (jnp.dot(ah, bh, preferred_element_type=f32)
            + (jnp.dot(al, bh, preferred_element_type=f32)
               + jnp.dot(ah, bl, preferred_element_type=f32)))


def _stage_kernel(nlayers, c1, acts, *refs):
    q_ref, kt_ref, p1_ref, p2_ref = refs[:4]
    wrefs = refs[4:4 + 2 * nlayers]
    out_ref = refs[-1]

    t = q_ref.shape[1]
    n2 = kt_ref.shape[2]

    q = q_ref[0, :, :]
    kt = kt_ref[0, :, :]
    aa = jnp.sum(q * q, axis=1, keepdims=True)
    bb = jnp.sum(kt * kt, axis=0, keepdims=True)
    ab = _mm(q, kt)
    d = jnp.maximum(aa + bb - 2.0 * ab, 0.0)

    iota = jax.lax.broadcasted_iota(jnp.int32, (t, n2), 1).astype(jnp.float32)
    big = jnp.float32(3.0e38)
    nf = jnp.float32(n2)
    dw = d
    wmat = None
    wtot = None
    for k in range(3):
        mn = jnp.min(dw, axis=1, keepdims=True)
        am = jnp.min(jnp.where(dw == mn, iota, nf), axis=1, keepdims=True)
        oh = iota == am
        wk = 1.0 / jnp.maximum(mn, 1e-10)
        wtot = wk if wtot is None else wtot + wk
        term = jnp.where(oh, wk, 0.0)
        wmat = term if wmat is None else wmat + term
        if k < 2:
            dw = jnp.where(oh, big, dw)

    interp = _mm3(wmat, p2_ref[0, :, :]) * (1.0 / wtot)

    w0 = wrefs[0]
    b0 = wrefs[1]
    x = (_mm(p1_ref[0, :, :], w0[:c1, :]) + _mm(interp, w0[c1:, :])
         + b0[0:1, :])
    x = jnp.maximum(x, 0.0) if acts[0] == 'r' else jnp.tanh(x)
    for i in range(1, nlayers):
        w = wrefs[2 * i]
        b = wrefs[2 * i + 1]
        x = _mm(x, w[:, :]) + b[0:1, :]
        x = jnp.maximum(x, 0.0) if acts[i] == 'r' else jnp.tanh(x)
    out_ref[0, :, :] = x


def _fp_stage(xyz1, xyz2, points1, points2, layers, acts, tile):
    bb, n1, _ = xyz1.shape
    n2 = xyz2.shape[1]
    c1 = points1.shape[2]
    c2 = points2.shape[2]
    out_dim = layers[-1][0].shape[1]
    kt = jnp.swapaxes(xyz2, 1, 2)

    args = [xyz1, kt, points1, points2]
    in_specs = [
        pl.BlockSpec((1, tile, 3), lambda b, s: (b, s, 0)),
        pl.BlockSpec((1, 3, n2), lambda b, s: (b, 0, 0)),
        pl.BlockSpec((1, tile, c1), lambda b, s: (b, s, 0)),
        pl.BlockSpec((1, n2, c2), lambda b, s: (b, 0, 0)),
    ]
    for w, b in layers:
        args += [w, b.reshape(1, -1)]
        in_specs.append(pl.BlockSpec(w.shape, lambda b, s: (0, 0)))
        in_specs.append(pl.BlockSpec((1, w.shape[1]), lambda b, s: (0, 0)))

    return pl.pallas_call(
        functools.partial(_stage_kernel, len(layers), c1, acts),
        grid=(bb, n1 // tile),
        in_specs=in_specs,
        out_specs=pl.BlockSpec((1, tile, out_dim), lambda b, s: (b, s, 0)),
        out_shape=jax.ShapeDtypeStruct((bb, n1, out_dim), jnp.float32),
        compiler_params=pltpu.CompilerParams(
            dimension_semantics=("parallel", "arbitrary")),
    )(*args)


def kernel(xyz0, xyz1, xyz2, xyz3, xyz4, points0, points1, points2, points3,
           points4, fa1_W0, fa1_b0, fa1_W1, fa1_b1, fa2_W0, fa2_b0, fa2_W1,
           fa2_b1, fa3_W0, fa3_b0, fa3_W1, fa3_b1, fa4_W0, fa4_b0, fa4_W1,
           fa4_b1, fa4_W2, fa4_b2, head_W0, head_b0, head_W1, head_b1,
           head_W2, head_b2):
    p3 = _fp_stage(xyz3, xyz4, points3, points4,
                   [(fa1_W0, fa1_b0), (fa1_W1, fa1_b1)], 'rr', 64)
    p2 = _fp_stage(xyz2, xyz3, points2, p3,
                   [(fa2_W0, fa2_b0), (fa2_W1, fa2_b1)], 'rr', 256)
    p1 = _fp_stage(xyz1, xyz2, points1, p2,
                   [(fa3_W0, fa3_b0), (fa3_W1, fa3_b1)], 'rr', 1024)
    head_W2p = jnp.pad(head_W2, ((0, 0), (0, 5)))
    head_b2p = jnp.pad(head_b2, (0, 5))
    out8 = _fp_stage(xyz0, xyz1, points0, p1,
                     [(fa4_W0, fa4_b0), (fa4_W1, fa4_b1), (fa4_W2, fa4_b2),
                      (head_W0, head_b0), (head_W1, head_b1),
                      (head_W2p, head_b2p)], 'rrrrrt', 512)
    return out8[..., :3]
